# baseline (speedup 1.0000x reference)
import functools

import jax
import jax.numpy as jnp
import numpy as np
from jax import lax
from jax.experimental import pallas as pl
from jax.experimental.pallas import tpu as pltpu
from jax.experimental.pallas import tpu_sc as plsc

DIM = 128
CAP = 100000
CAP_PAD = 100352
CB = 1024
B = 1024
K = 50
DELTA = 1e-3
EPS = 1e-12

NV = CAP // 16
CANDS = 256
REBUILD_AT = CANDS - 16
NQ_PER_TEC = B // 32
BIG_I32 = np.int32(2**30)


def _sims_body(q_ref, k_ref, sims_ref, kn1_ref, kn2_ref, qn1_ref, qn2_ref):
    q = q_ref[...]
    qs = q * q
    qn2c = jnp.sum(qs, axis=1, keepdims=True)
    qn = q / (jnp.sqrt(qn2c) + EPS)
    k = k_ref[...]
    ks = k * k
    kn2c = jnp.sum(ks, axis=1, keepdims=True)
    kn = k / (jnp.sqrt(kn2c) + EPS)
    sims_ref[...] = lax.dot_general(
        qn, kn, (((1,), (1,)), ((), ())), preferred_element_type=jnp.float32
    )
    ones = jnp.ones((1, DIM), jnp.float32)
    kn2r = lax.dot_general(
        ones, ks, (((1,), (1,)), ((), ())),
        precision=lax.Precision.HIGHEST, preferred_element_type=jnp.float32,
    )
    kn1_ref[...] = (jnp.sqrt(kn2r) + EPS).reshape(1, 1, CB)
    kn2_ref[...] = kn2r.reshape(1, 1, CB)

    @pl.when(pl.program_id(0) == 0)
    def _():
        qn2r = lax.dot_general(
            ones, qs, (((1,), (1,)), ((), ())),
            precision=lax.Precision.HIGHEST, preferred_element_type=jnp.float32,
        )
        qn1_ref[...] = jnp.sqrt(qn2r) + EPS
        qn2_ref[...] = qn2r


_sims = pl.pallas_call(
    _sims_body,
    grid=(CAP_PAD // CB,),
    in_specs=[
        pl.BlockSpec((B, DIM), lambda i: (0, 0)),
        pl.BlockSpec((CB, DIM), lambda i: (i, 0)),
    ],
    out_specs=[
        pl.BlockSpec((B, CB), lambda i: (0, i)),
        pl.BlockSpec((1, 1, CB), lambda i: (i, 0, 0)),
        pl.BlockSpec((1, 1, CB), lambda i: (i, 0, 0)),
        pl.BlockSpec((1, B), lambda i: (0, 0)),
        pl.BlockSpec((1, B), lambda i: (0, 0)),
    ],
    out_shape=[
        jax.ShapeDtypeStruct((B, CAP_PAD), jnp.float32),
        jax.ShapeDtypeStruct((CAP_PAD // CB, 1, CB), jnp.float32),
        jax.ShapeDtypeStruct((CAP_PAD // CB, 1, CB), jnp.float32),
        jax.ShapeDtypeStruct((1, B), jnp.float32),
        jax.ShapeDtypeStruct((1, B), jnp.float32),
    ],
)


_sc_mesh = plsc.VectorSubcoreMesh(core_axis_name="c", subcore_axis_name="s")


@functools.partial(
    pl.kernel,
    out_type=jax.ShapeDtypeStruct((B,), jnp.float32),
    mesh=_sc_mesh,
    compiler_params=pltpu.CompilerParams(needs_layout_passes=False),
    scratch_types=[
        pltpu.VMEM((CAP_PAD,), jnp.float32),
        pltpu.VMEM((CANDS,), jnp.float32),
        pltpu.VMEM((CANDS,), jnp.int32),
        pltpu.VMEM((64,), jnp.float32),
        pltpu.VMEM((64,), jnp.int32),
        pltpu.VMEM((64, 128), jnp.float32),
        pltpu.VMEM((NQ_PER_TEC,), jnp.float32),
        pltpu.VMEM((NQ_PER_TEC,), jnp.float32),
        pltpu.VMEM((NQ_PER_TEC,), jnp.float32),
        pltpu.VMEM((16,), jnp.int32),
        pltpu.SemaphoreType.DMA,
        pltpu.SemaphoreType.DMA,
        pltpu.SemaphoreType.DMA,
    ],
)
def _sc_select(sims_hbm, aux_hbm, qn1_hbm, qn2_hbm, out_hbm,
               row_v, cval, cidx, ext_v, ext_i, aux_v, q1v, q2v, outv, flagbuf,
               sem, semA, semB):
    wid = lax.axis_index("s") * 2 + lax.axis_index("c")
    qbase = wid * NQ_PER_TEC
    pltpu.sync_copy(qn1_hbm.at[pl.ds(qbase, NQ_PER_TEC)], q1v)
    pltpu.sync_copy(qn2_hbm.at[pl.ds(qbase, NQ_PER_TEC)], q2v)

    iota = lax.iota(jnp.int32, 16)
    run_base = iota * 16
    neg_inf = jnp.full((16,), -jnp.inf, jnp.float32)
    zeros_i = jnp.zeros((16,), jnp.int32)
    lane0 = iota == 0

    def extract_topk(n):
        ext_v[pl.ds(48, 16)] = neg_inf
        ext_i[pl.ds(48, 16)] = zeros_i
        for kk in range(CANDS // 16):
            g = iota + kk * 16
            sl = cval[pl.ds(kk * 16, 16)]
            cval[pl.ds(kk * 16, 16)] = jnp.where(g < n, sl, neg_inf)
        for kk in range(CANDS // 16):
            sv, si = plsc.sort_key_val(
                cval[pl.ds(kk * 16, 16)], cidx[pl.ds(kk * 16, 16)],
                descending=True,
            )
            cval[pl.ds(kk * 16, 16)] = sv
            cidx[pl.ds(kk * 16, 16)] = si

        def merge_body(j, carry):
            headoff, _ = carry
            pos = jnp.minimum(run_base + headoff, CANDS - 1)
            hv = plsc.load_gather(cval, [pos])
            hv = jnp.where(headoff < 16, hv, neg_inf)
            m = jnp.max(hv)
            hi = plsc.load_gather(cidx, [pos])
            eq = hv == m
            mi = jnp.min(jnp.where(eq, hi, BIG_I32))
            sel = eq & (hi == mi)
            headoff = headoff + sel.astype(jnp.int32)
            j_sp = lax.broadcast(j, (16,))
            plsc.store_scatter(ext_v, [j_sp], lax.broadcast(m, (16,)), mask=lane0)
            plsc.store_scatter(ext_i, [j_sp], lax.broadcast(mi, (16,)), mask=lane0)
            return headoff, m

        _, t_new = lax.fori_loop(
            0, K, merge_body, (zeros_i, jnp.float32(0.0))
        )
        return t_new

    ones_i = zeros_i + 1

    def refill_from_ext():
        for r in range(4):
            cval[pl.ds(r * 16, 16)] = ext_v[pl.ds(r * 16, 16)]
            cidx[pl.ds(r * 16, 16)] = ext_i[pl.ds(r * 16, 16)]

    lane_masks = [iota == j for j in range(16)]

    HALF = CAP_PAD // 2

    def _half0(qg):
        return pltpu.make_async_copy(
            sims_hbm.at[qg, pl.ds(0, HALF)], row_v.at[pl.ds(0, HALF)], semA
        )

    def _half1(qg):
        return pltpu.make_async_copy(
            sims_hbm.at[qg, pl.ds(HALF, HALF)], row_v.at[pl.ds(HALF, HALF)],
            semB,
        )

    _half0(qbase).start()

    def process_query(qi, _):
        qg = qbase + qi
        _half0(qg).wait()
        _half1(qg).start()

        for kk in range(CANDS // 16):
            cval[pl.ds(kk * 16, 16)] = row_v[pl.ds(kk * 16, 16)]
            cidx[pl.ds(kk * 16, 16)] = iota + kk * 16
        t0 = extract_topk(jnp.int32(CANDS))
        refill_from_ext()

        def sweep(base, fl, n, tv):
            s0 = jnp.max(plsc.all_reduce_population_count(fl > zeros_i))

            def cond_fun(carry2):
                return carry2[0] > 0

            def sweep_body(carry2):
                s, n, tv, fl = carry2
                j_sp = plsc.all_reduce_ffs(fl > zeros_i)
                fl = jnp.where(iota == j_sp, zeros_i, fl)
                idxv = j_sp * 16 + iota + base
                v = plsc.load_gather(row_v, [idxv])
                msk = v > tv
                cnt = jnp.max(plsc.all_reduce_population_count(msk))
                plsc.store_compressed(cval.at[pl.ds(n, 16)], v, mask=msk)
                plsc.store_compressed(cidx.at[pl.ds(n, 16)], idxv, mask=msk)
                n2 = n + cnt

                def do_rebuild():
                    t_new = extract_topk(n2)
                    refill_from_ext()
                    return jnp.int32(K), lax.broadcast(t_new, (16,))

                n3, tv3 = lax.cond(n2 >= REBUILD_AT, do_rebuild,
                                   lambda: (n2, tv))
                return s - 1, n3, tv3, fl

            _, n_f, tv_f, _ = lax.while_loop(
                cond_fun, sweep_body, (s0, n, tv, fl)
            )
            return n_f, tv_f

        def half_flags(base, tv):
            fl = zeros_i
            for j in range(16):
                v = row_v[pl.ds(base + j * 16, 16)]
                msk = v > tv
                c = plsc.all_reduce_population_count(msk)
                fl = jnp.where(lane_masks[j], c, fl)
            return fl

        tv0 = lax.broadcast(t0, (16,))
        fl0 = half_flags(256, tv0)
        n1, tv1 = lax.cond(
            jnp.max(fl0) > 0, lambda: sweep(256, fl0, jnp.int32(K), tv0),
            lambda: (jnp.int32(K), tv0),
        )

        def group_body(g, carry):
            n, tv = carry
            base = g * 512
            mx = row_v[pl.ds(base, 16)]
            for j in range(1, 32):
                mx = jnp.maximum(mx, row_v[pl.ds(base + j * 16, 16)])
            any_c = jnp.max(plsc.all_reduce_population_count(mx > tv))

            def both():
                fla = half_flags(base, tv)
                flb = half_flags(base + 256, tv)
                n2, tv2 = sweep(base, fla, n, tv)
                return sweep(base + 256, flb, n2, tv2)

            return lax.cond(any_c > 0, both, lambda: (n, tv))

        nh, tvh = lax.fori_loop(1, HALF // 512, group_body,
                                (n1, tv1), unroll=2)

        _half1(qg).wait()
        _half0(jnp.minimum(qg + 1, B - 1)).start()
        for kk in range((CAP_PAD - CAP) // 16):
            row_v[pl.ds(CAP + kk * 16, 16)] = neg_inf

        n_f, _ = lax.fori_loop(HALF // 512, CAP_PAD // 512, group_body,
                               (nh, tvh), unroll=2)

        extract_topk(n_f)
        pltpu.async_copy(aux_hbm.at[ext_i], aux_v, sem).wait()

        qi_sp = lax.broadcast(qi, (16,))
        q1 = plsc.load_gather(q1v, [qi_sp])
        q2 = plsc.load_gather(q2v, [qi_sp])
        accw = jnp.zeros((16,), jnp.float32)
        accwv = jnp.zeros((16,), jnp.float32)
        for r in range(4):
            rows = iota + r * 16
            cos = ext_v[pl.ds(r * 16, 16)]
            k1 = plsc.load_gather(aux_v, [rows, zeros_i])
            k2 = plsc.load_gather(aux_v, [rows, zeros_i + 1])
            vv = plsc.load_gather(aux_v, [rows, zeros_i + 2])
            dot = cos * (q1 * k1)
            d2 = q2 + k2 - 2.0 * dot
            w = 1.0 / (d2 + DELTA)
            accw = accw + w
            accwv = accwv + w * vv
        res = lax.broadcast(jnp.sum(accwv), (16,)) / lax.broadcast(jnp.sum(accw), (16,))
        plsc.store_scatter(outv, [qi_sp], res, mask=lane0)
        return 0

    lax.fori_loop(0, NQ_PER_TEC, process_query, 0)
    _half0(qbase).wait()
    pltpu.sync_copy(outv, out_hbm.at[pl.ds(qbase, NQ_PER_TEC)])


def kernel(key, keys_table, values_table):
    kp = jnp.pad(keys_table, ((0, CAP_PAD - CAP), (0, 0)))
    sims, kn1, kn2, qn1, qn2 = _sims(key, kp)
    vpad = jnp.pad(values_table, (0, CAP_PAD - CAP))
    aux = jnp.concatenate(
        [
            kn1.reshape(CAP_PAD, 1),
            kn2.reshape(CAP_PAD, 1),
            vpad.reshape(CAP_PAD, 1),
            jnp.zeros((CAP_PAD, 125), jnp.float32),
        ],
        axis=1,
    )
    return _sc_select(sims, aux, qn1.reshape(B), qn2.reshape(B))


# speedup vs baseline: 1.1112x; 1.1112x over previous
import functools

import jax
import jax.numpy as jnp
import numpy as np
from jax import lax
from jax.experimental import pallas as pl
from jax.experimental.pallas import tpu as pltpu
from jax.experimental.pallas import tpu_sc as plsc

DIM = 128
CAP = 100000
CAP_PAD = 100352
CB = 1024
B = 1024
K = 50
DELTA = 1e-3
EPS = 1e-12

NV = CAP // 16
CANDS = 256
REBUILD_AT = CANDS - 16
NQ_PER_TEC = B // 32
BIG_I32 = np.int32(2**30)


def _sims_body(q_ref, k_ref, sims_ref, kn1_ref, kn2_ref, qn1_ref, qn2_ref):
    q = q_ref[...]
    qs = q * q
    qn2c = jnp.sum(qs, axis=1, keepdims=True)
    qn = q / (jnp.sqrt(qn2c) + EPS)
    k = k_ref[...]
    ks = k * k
    kn2c = jnp.sum(ks, axis=1, keepdims=True)
    kn = k / (jnp.sqrt(kn2c) + EPS)
    sims_ref[...] = lax.dot_general(
        qn, kn, (((1,), (1,)), ((), ())), preferred_element_type=jnp.float32
    )
    ones = jnp.ones((1, DIM), jnp.float32)
    kn2r = lax.dot_general(
        ones, ks, (((1,), (1,)), ((), ())),
        precision=lax.Precision.HIGHEST, preferred_element_type=jnp.float32,
    )
    kn1_ref[...] = (jnp.sqrt(kn2r) + EPS).reshape(1, 1, CB)
    kn2_ref[...] = kn2r.reshape(1, 1, CB)

    @pl.when(pl.program_id(0) == 0)
    def _():
        qn2r = lax.dot_general(
            ones, qs, (((1,), (1,)), ((), ())),
            precision=lax.Precision.HIGHEST, preferred_element_type=jnp.float32,
        )
        qn1_ref[...] = jnp.sqrt(qn2r) + EPS
        qn2_ref[...] = qn2r


_sims = pl.pallas_call(
    _sims_body,
    grid=(CAP_PAD // CB,),
    in_specs=[
        pl.BlockSpec((B, DIM), lambda i: (0, 0)),
        pl.BlockSpec((CB, DIM), lambda i: (i, 0)),
    ],
    out_specs=[
        pl.BlockSpec((B, CB), lambda i: (0, i)),
        pl.BlockSpec((1, 1, CB), lambda i: (i, 0, 0)),
        pl.BlockSpec((1, 1, CB), lambda i: (i, 0, 0)),
        pl.BlockSpec((1, B), lambda i: (0, 0)),
        pl.BlockSpec((1, B), lambda i: (0, 0)),
    ],
    out_shape=[
        jax.ShapeDtypeStruct((B, CAP_PAD), jnp.float32),
        jax.ShapeDtypeStruct((CAP_PAD // CB, 1, CB), jnp.float32),
        jax.ShapeDtypeStruct((CAP_PAD // CB, 1, CB), jnp.float32),
        jax.ShapeDtypeStruct((1, B), jnp.float32),
        jax.ShapeDtypeStruct((1, B), jnp.float32),
    ],
)


_sc_mesh = plsc.VectorSubcoreMesh(core_axis_name="c", subcore_axis_name="s")


@functools.partial(
    pl.kernel,
    out_type=jax.ShapeDtypeStruct((B,), jnp.float32),
    mesh=_sc_mesh,
    compiler_params=pltpu.CompilerParams(needs_layout_passes=False),
    scratch_types=[
        pltpu.VMEM((CAP_PAD,), jnp.float32),
        pltpu.VMEM((CANDS,), jnp.float32),
        pltpu.VMEM((CANDS,), jnp.int32),
        pltpu.VMEM((64,), jnp.float32),
        pltpu.VMEM((64,), jnp.int32),
        pltpu.VMEM((64, 128), jnp.float32),
        pltpu.VMEM((NQ_PER_TEC,), jnp.float32),
        pltpu.VMEM((NQ_PER_TEC,), jnp.float32),
        pltpu.VMEM((NQ_PER_TEC,), jnp.float32),
        pltpu.VMEM((16,), jnp.int32),
        pltpu.SemaphoreType.DMA,
        pltpu.SemaphoreType.DMA,
        pltpu.SemaphoreType.DMA,
    ],
)
def _sc_select(sims_hbm, aux_hbm, qn1_hbm, qn2_hbm, out_hbm,
               row_v, cval, cidx, ext_v, ext_i, aux_v, q1v, q2v, outv, flagbuf,
               sem, semA, semB):
    wid = lax.axis_index("s") * 2 + lax.axis_index("c")
    qbase = wid * NQ_PER_TEC
    pltpu.sync_copy(qn1_hbm.at[pl.ds(qbase, NQ_PER_TEC)], q1v)
    pltpu.sync_copy(qn2_hbm.at[pl.ds(qbase, NQ_PER_TEC)], q2v)

    iota = lax.iota(jnp.int32, 16)
    run_base = iota * 16
    neg_inf = jnp.full((16,), -jnp.inf, jnp.float32)
    zeros_i = jnp.zeros((16,), jnp.int32)
    lane0 = iota == 0

    def extract_topk(n):
        ext_v[pl.ds(48, 16)] = neg_inf
        ext_i[pl.ds(48, 16)] = zeros_i
        for kk in range(CANDS // 16):
            g = iota + kk * 16
            sl = cval[pl.ds(kk * 16, 16)]
            cval[pl.ds(kk * 16, 16)] = jnp.where(g < n, sl, neg_inf)
        for kk in range(CANDS // 16):
            sv, si = plsc.sort_key_val(
                cval[pl.ds(kk * 16, 16)], cidx[pl.ds(kk * 16, 16)],
                descending=True,
            )
            cval[pl.ds(kk * 16, 16)] = sv
            cidx[pl.ds(kk * 16, 16)] = si

        def merge_body(j, carry):
            headoff, _ = carry
            pos = jnp.minimum(run_base + headoff, CANDS - 1)
            hv = plsc.load_gather(cval, [pos])
            hv = jnp.where(headoff < 16, hv, neg_inf)
            m = jnp.max(hv)
            hi = plsc.load_gather(cidx, [pos])
            eq = hv == m
            sel = iota == plsc.all_reduce_ffs(eq)
            headoff = headoff + jnp.where(sel, 1, 0)
            j_sp = lax.broadcast(j, (16,))
            plsc.store_scatter(ext_v, [j_sp], hv, mask=sel)
            plsc.store_scatter(ext_i, [j_sp], hi, mask=sel)
            return headoff, m

        _, t_new = lax.fori_loop(
            0, K, merge_body, (zeros_i, jnp.float32(0.0))
        )
        return t_new

    ones_i = zeros_i + 1

    def refill_from_ext():
        for r in range(4):
            cval[pl.ds(r * 16, 16)] = ext_v[pl.ds(r * 16, 16)]
            cidx[pl.ds(r * 16, 16)] = ext_i[pl.ds(r * 16, 16)]

    lane_masks = [iota == j for j in range(16)]

    HALF = CAP_PAD // 2

    def _half0(qg):
        return pltpu.make_async_copy(
            sims_hbm.at[qg, pl.ds(0, HALF)], row_v.at[pl.ds(0, HALF)], semA
        )

    def _half1(qg):
        return pltpu.make_async_copy(
            sims_hbm.at[qg, pl.ds(HALF, HALF)], row_v.at[pl.ds(HALF, HALF)],
            semB,
        )

    _half0(qbase).start()

    def process_query(qi, _):
        qg = qbase + qi
        _half0(qg).wait()
        _half1(qg).start()

        for kk in range(CANDS // 16):
            cval[pl.ds(kk * 16, 16)] = row_v[pl.ds(kk * 16, 16)]
            cidx[pl.ds(kk * 16, 16)] = iota + kk * 16
        t0 = extract_topk(jnp.int32(CANDS))
        refill_from_ext()

        def sweep(base, fl, n, tv):
            s0 = jnp.max(plsc.all_reduce_population_count(fl > zeros_i))

            def cond_fun(carry2):
                return carry2[0] > 0

            def sweep_body(carry2):
                s, n, tv, fl = carry2
                j_sp = plsc.all_reduce_ffs(fl > zeros_i)
                fl = jnp.where(iota == j_sp, zeros_i, fl)
                idxv = j_sp * 16 + iota + base
                v = plsc.load_gather(row_v, [idxv])
                msk = v > tv
                cnt = jnp.max(plsc.all_reduce_population_count(msk))
                plsc.store_compressed(cval.at[pl.ds(n, 16)], v, mask=msk)
                plsc.store_compressed(cidx.at[pl.ds(n, 16)], idxv, mask=msk)
                n2 = n + cnt

                def do_rebuild():
                    t_new = extract_topk(n2)
                    refill_from_ext()
                    return jnp.int32(K), lax.broadcast(t_new, (16,))

                n3, tv3 = lax.cond(n2 >= REBUILD_AT, do_rebuild,
                                   lambda: (n2, tv))
                return s - 1, n3, tv3, fl

            _, n_f, tv_f, _ = lax.while_loop(
                cond_fun, sweep_body, (s0, n, tv, fl)
            )
            return n_f, tv_f

        def half_flags(base, tv):
            fl = zeros_i
            for j in range(16):
                v = row_v[pl.ds(base + j * 16, 16)]
                msk = v > tv
                c = plsc.all_reduce_population_count(msk)
                fl = jnp.where(lane_masks[j], c, fl)
            return fl

        tv0 = lax.broadcast(t0, (16,))
        fl0 = half_flags(256, tv0)
        n1, tv1 = lax.cond(
            jnp.max(fl0) > 0, lambda: sweep(256, fl0, jnp.int32(K), tv0),
            lambda: (jnp.int32(K), tv0),
        )

        def group_body(g, carry):
            n, tv = carry
            base = g * 512
            fla = half_flags(base, tv)
            flb = half_flags(base + 256, tv)
            any_c = jnp.max(jnp.maximum(fla, flb))

            def both():
                n2, tv2 = sweep(base, fla, n, tv)
                return sweep(base + 256, flb, n2, tv2)

            return lax.cond(any_c > 0, both, lambda: (n, tv))

        nh, tvh = lax.fori_loop(1, HALF // 512, group_body,
                                (n1, tv1), unroll=4)

        _half1(qg).wait()
        _half0(jnp.minimum(qg + 1, B - 1)).start()
        for kk in range((CAP_PAD - CAP) // 16):
            row_v[pl.ds(CAP + kk * 16, 16)] = neg_inf

        n_f, _ = lax.fori_loop(HALF // 512, CAP_PAD // 512, group_body,
                               (nh, tvh), unroll=4)

        extract_topk(n_f)
        pltpu.async_copy(aux_hbm.at[ext_i], aux_v, sem).wait()

        qi_sp = lax.broadcast(qi, (16,))
        q1 = plsc.load_gather(q1v, [qi_sp])
        q2 = plsc.load_gather(q2v, [qi_sp])
        accw = jnp.zeros((16,), jnp.float32)
        accwv = jnp.zeros((16,), jnp.float32)
        for r in range(4):
            rows = iota + r * 16
            cos = ext_v[pl.ds(r * 16, 16)]
            k1 = plsc.load_gather(aux_v, [rows, zeros_i])
            k2 = plsc.load_gather(aux_v, [rows, zeros_i + 1])
            vv = plsc.load_gather(aux_v, [rows, zeros_i + 2])
            dot = cos * (q1 * k1)
            d2 = q2 + k2 - 2.0 * dot
            w = 1.0 / (d2 + DELTA)
            accw = accw + w
            accwv = accwv + w * vv
        res = lax.broadcast(jnp.sum(accwv), (16,)) / lax.broadcast(jnp.sum(accw), (16,))
        plsc.store_scatter(outv, [qi_sp], res, mask=lane0)
        return 0

    lax.fori_loop(0, NQ_PER_TEC, process_query, 0)
    _half0(qbase).wait()
    pltpu.sync_copy(outv, out_hbm.at[pl.ds(qbase, NQ_PER_TEC)])


def kernel(key, keys_table, values_table):
    kp = jnp.pad(keys_table, ((0, CAP_PAD - CAP), (0, 0)))
    sims, kn1, kn2, qn1, qn2 = _sims(key, kp)
    vpad = jnp.pad(values_table, (0, CAP_PAD - CAP))
    aux = jnp.concatenate(
        [
            kn1.reshape(CAP_PAD, 1),
            kn2.reshape(CAP_PAD, 1),
            vpad.reshape(CAP_PAD, 1),
            jnp.zeros((CAP_PAD, 125), jnp.float32),
        ],
        axis=1,
    )
    return _sc_select(sims, aux, qn1.reshape(B), qn2.reshape(B))
